# baseline (speedup 1.0000x reference)
import functools
import jax
import jax.numpy as jnp
from jax import lax
from jax.experimental import pallas as pl
from jax.experimental.pallas import tpu as pltpu
from jax.experimental.pallas import tpu_sc as plsc


def _make_sc_bcast(S, N, E):
    info = plsc.get_sparse_core_info()
    nw = info.num_cores * info.num_subcores
    rows_per_w = S // nw
    mesh = plsc.VectorSubcoreMesh(core_axis_name="c", subcore_axis_name="s")

    @functools.partial(
        pl.kernel,
        out_type=jax.ShapeDtypeStruct((S, N * E), jnp.float32),
        mesh=mesh,
        scratch_types=[pltpu.VMEM((rows_per_w, E), jnp.float32)],
    )
    def sc_bcast(table_hbm, out_hbm, buf):
        wid = lax.axis_index("s") * info.num_cores + lax.axis_index("c")
        r0 = wid * rows_per_w
        pltpu.sync_copy(table_hbm.at[pl.ds(r0, rows_per_w)], buf)
        for n in range(N):
            pltpu.sync_copy(buf, out_hbm.at[pl.ds(r0, rows_per_w), pl.ds(n * E, E)])

    return sc_bcast


def kernel(x, table):
    S, N = x.shape
    _, E = table.shape
    out2d = _make_sc_bcast(S, N, E)(table)
    return out2d.reshape(S, N, E)


# speedup vs baseline: 1.0058x; 1.0058x over previous
import functools
import jax
import jax.numpy as jnp
from jax import lax
from jax.experimental import pallas as pl
from jax.experimental.pallas import tpu as pltpu
from jax.experimental.pallas import tpu_sc as plsc


def _make_sc_bcast(S, N, E):
    info = plsc.get_sparse_core_info()
    nw = info.num_cores * info.num_subcores
    rows_per_w = S // nw
    mesh = plsc.VectorSubcoreMesh(core_axis_name="c", subcore_axis_name="s")

    @functools.partial(
        pl.kernel,
        out_type=jax.ShapeDtypeStruct((S, N * E), jnp.float32),
        mesh=mesh,
        scratch_types=[
            pltpu.VMEM((rows_per_w, E), jnp.float32),
            pltpu.SemaphoreType.DMA,
        ],
    )
    def sc_bcast(table_hbm, out_hbm, buf, sem):
        wid = lax.axis_index("s") * info.num_cores + lax.axis_index("c")
        r0 = wid * rows_per_w
        pltpu.sync_copy(table_hbm.at[pl.ds(r0, rows_per_w)], buf)
        copies = [
            pltpu.async_copy(
                buf, out_hbm.at[pl.ds(r0, rows_per_w), pl.ds(n * E, E)], sem
            )
            for n in range(N)
        ]
        for c in copies:
            c.wait()

    return sc_bcast


def kernel(x, table):
    S, N = x.shape
    _, E = table.shape
    out2d = _make_sc_bcast(S, N, E)(table)
    return out2d.reshape(S, N, E)


# speedup vs baseline: 2.2557x; 2.2428x over previous
import functools
import jax
import jax.numpy as jnp
from jax import lax
from jax.experimental import pallas as pl
from jax.experimental.pallas import tpu as pltpu
from jax.experimental.pallas import tpu_sc as plsc


def _make_sc_bcast(S, N, E):
    info = plsc.get_sparse_core_info()
    nw = info.num_cores * info.num_subcores
    rows_per_w = S // nw
    mesh = plsc.VectorSubcoreMesh(core_axis_name="c", subcore_axis_name="s")

    @functools.partial(
        pl.kernel,
        out_type=jax.ShapeDtypeStruct((S, N, E), jnp.float32),
        mesh=mesh,
        scratch_types=[
            pltpu.VMEM((rows_per_w, E), jnp.float32),
            pltpu.SemaphoreType.DMA,
        ],
    )
    def sc_bcast(table_hbm, out_hbm, buf, sem):
        wid = lax.axis_index("s") * info.num_cores + lax.axis_index("c")
        r0 = wid * rows_per_w
        pltpu.sync_copy(table_hbm.at[pl.ds(r0, rows_per_w)], buf)
        copies = [
            pltpu.async_copy(buf, out_hbm.at[pl.ds(r0, rows_per_w), n], sem)
            for n in range(N)
        ]
        for c in copies:
            c.wait()

    return sc_bcast


def kernel(x, table):
    S, N = x.shape
    _, E = table.shape
    return _make_sc_bcast(S, N, E)(table)
